# baseline (speedup 1.0000x reference)
import functools

import jax
import jax.numpy as jnp
from jax import lax
from jax.experimental import pallas as pl
from jax.experimental.pallas import tpu as pltpu
from jax.experimental.pallas import tpu_sc as plsc

_ROWS = 200
_COLS = 16384
_TSIZE = 93
_NC = 2
_NS = 16
_NW = _NC * _NS
_COLS_W = _COLS // _NW
_CCHUNK = 128
_NCHUNK = _COLS_W // _CCHUNK
_LANES = 16

_mesh = plsc.VectorSubcoreMesh(core_axis_name="c", subcore_axis_name="s")


def _make(interpret=False):
    return functools.partial(
        pl.kernel,
        mesh=_mesh,
        out_type=jax.ShapeDtypeStruct((_ROWS, _COLS), jnp.int32),
        compiler_params=pltpu.CompilerParams(needs_layout_passes=False),
        scratch_types=[
            pltpu.VMEM((_TSIZE,), jnp.int32),
            pltpu.VMEM((_NCHUNK, _ROWS, _CCHUNK), jnp.int32),
            pltpu.SemaphoreType.DMA((_NCHUNK,)),
            pltpu.SemaphoreType.DMA((_NCHUNK,)),
        ],
        interpret=interpret,
    )


def _lookup_body(values_hbm, table_hbm, out_hbm, table_v, buf, insem, outsem):
    wid = lax.axis_index("s") * _NC + lax.axis_index("c")
    c0w = wid * _COLS_W

    def in_slice(g):
        return values_hbm.at[pl.ds(0, _ROWS),
                             pl.ds(c0w + g * _CCHUNK, _CCHUNK)]

    def out_slice(g):
        return out_hbm.at[pl.ds(0, _ROWS), pl.ds(c0w + g * _CCHUNK, _CCHUNK)]

    pltpu.sync_copy(table_hbm, table_v)
    for g in range(_NCHUNK):
        pltpu.async_copy(in_slice(g), buf.at[g], insem.at[g])

    for g in range(_NCHUNK):
        pltpu.make_async_copy(in_slice(g), buf.at[g], insem.at[g]).wait()
        bg = buf.at[g]

        @plsc.parallel_loop(0, _ROWS, unroll=1)
        def _(r):
            for c0 in range(0, _CCHUNK, _LANES):
                idx = bg[r, pl.ds(c0, _LANES)]
                bg[r, pl.ds(c0, _LANES)] = plsc.load_gather(table_v, [idx])

        pltpu.async_copy(buf.at[g], out_slice(g), outsem.at[g])

    for g in range(_NCHUNK):
        pltpu.make_async_copy(buf.at[g], out_slice(g), outsem.at[g]).wait()


_lookup = _make()(_lookup_body)


def kernel(values, table):
    out_t = _lookup(jnp.swapaxes(values, 0, 1), table)
    return jnp.swapaxes(out_t, 0, 1)


# speedup vs baseline: 1.0681x; 1.0681x over previous
import functools

import jax
import jax.numpy as jnp
from jax import lax
from jax.experimental import pallas as pl
from jax.experimental.pallas import tpu as pltpu
from jax.experimental.pallas import tpu_sc as plsc

_ROWS = 200
_COLS = 16384
_TSIZE = 93
_NC = 2
_NS = 16
_NW = _NC * _NS
_COLS_W = _COLS // _NW
_CCHUNK = 128
_NCHUNK = _COLS_W // _CCHUNK
_NB = 2
_LANES = 16

_mesh = plsc.VectorSubcoreMesh(core_axis_name="c", subcore_axis_name="s")


def _make(interpret=False):
    return functools.partial(
        pl.kernel,
        mesh=_mesh,
        out_type=jax.ShapeDtypeStruct((_ROWS, _COLS), jnp.int32),
        compiler_params=pltpu.CompilerParams(needs_layout_passes=False),
        scratch_types=[
            pltpu.VMEM((_TSIZE,), jnp.int32),
            pltpu.VMEM((_NB, _ROWS, _CCHUNK), jnp.int32),
            pltpu.VMEM((_NB, _ROWS, _CCHUNK), jnp.int32),
            pltpu.SemaphoreType.DMA((_NB,)),
            pltpu.SemaphoreType.DMA((_NB,)),
        ],
        interpret=interpret,
    )


def _lookup_body(values_hbm, table_hbm, out_hbm, table_v, in_v, out_v, insem,
                 outsem):
    wid = lax.axis_index("s") * _NC + lax.axis_index("c")
    c0w = wid * _COLS_W
    pltpu.sync_copy(table_hbm, table_v)

    def gather_chunk(b):
        ib = in_v.at[b]
        ob = out_v.at[b]

        @plsc.parallel_loop(0, _ROWS, unroll=1)
        def _(r):
            for c0 in range(0, _CCHUNK, _LANES):
                idx = ib[r, pl.ds(c0, _LANES)]
                ob[r, pl.ds(c0, _LANES)] = plsc.load_gather(table_v, [idx])

    def in_slice(g):
        return values_hbm.at[pl.ds(0, _ROWS),
                             pl.ds(c0w + g * _CCHUNK, _CCHUNK)]

    def out_slice(g):
        return out_hbm.at[pl.ds(0, _ROWS), pl.ds(c0w + g * _CCHUNK, _CCHUNK)]

    for b in range(_NB):
        pltpu.async_copy(in_slice(b), in_v.at[b], insem.at[b])

    for b in range(_NB):
        pltpu.make_async_copy(in_slice(b), in_v.at[b], insem.at[b]).wait()
        gather_chunk(b)
        pltpu.async_copy(out_v.at[b], out_slice(b), outsem.at[b])
        if b + _NB < _NCHUNK:
            pltpu.async_copy(in_slice(b + _NB), in_v.at[b], insem.at[b])

    @pl.loop(_NB, _NCHUNK, step=_NB)
    def _(g0):
        for b in range(_NB):
            g = g0 + b
            pltpu.make_async_copy(in_slice(g), in_v.at[b], insem.at[b]).wait()
            pltpu.make_async_copy(out_v.at[b], out_slice(g),
                                  outsem.at[b]).wait()
            gather_chunk(b)
            pltpu.async_copy(out_v.at[b], out_slice(g), outsem.at[b])

            @pl.when(g + _NB < _NCHUNK)
            def _():
                pltpu.async_copy(in_slice(g + _NB), in_v.at[b], insem.at[b])

    for b in range(_NB):
        pltpu.make_async_copy(out_v.at[b], out_slice(_NCHUNK - _NB + b),
                              outsem.at[b]).wait()


_lookup = _make()(_lookup_body)


def kernel(values, table):
    out_t = _lookup(jnp.swapaxes(values, 0, 1), table)
    return jnp.swapaxes(out_t, 0, 1)


# speedup vs baseline: 1.0872x; 1.0179x over previous
import functools

import jax
import jax.numpy as jnp
from jax import lax
from jax.experimental import pallas as pl
from jax.experimental.pallas import tpu as pltpu
from jax.experimental.pallas import tpu_sc as plsc

_ROWS = 200
_COLS = 16384
_TSIZE = 93
_NC = 2
_NS = 16
_NW = _NC * _NS
_COLS_W = _COLS // _NW
_CCHUNK = 128
_RCHUNK = 40
_NRC = _ROWS // _RCHUNK
_NCHUNK = (_COLS_W // _CCHUNK) * _NRC
_NB = 4
_LANES = 16

_mesh = plsc.VectorSubcoreMesh(core_axis_name="c", subcore_axis_name="s")


def _make(interpret=False):
    return functools.partial(
        pl.kernel,
        mesh=_mesh,
        out_type=jax.ShapeDtypeStruct((_ROWS, _COLS), jnp.int32),
        compiler_params=pltpu.CompilerParams(needs_layout_passes=False),
        scratch_types=[
            pltpu.VMEM((_TSIZE,), jnp.int32),
            pltpu.VMEM((_NB, _RCHUNK, _CCHUNK), jnp.int32),
            pltpu.VMEM((_NB, _RCHUNK, _CCHUNK), jnp.int32),
            pltpu.SemaphoreType.DMA((_NB,)),
            pltpu.SemaphoreType.DMA((_NB,)),
        ],
        interpret=interpret,
    )


def _lookup_body(values_hbm, table_hbm, out_hbm, table_v, in_v, out_v, insem,
                 outsem):
    wid = lax.axis_index("s") * _NC + lax.axis_index("c")
    c0w = wid * _COLS_W
    pltpu.sync_copy(table_hbm, table_v)

    def gather_chunk(b):
        ib = in_v.at[b]
        ob = out_v.at[b]

        @plsc.parallel_loop(0, _RCHUNK, unroll=1)
        def _(r):
            for c0 in range(0, _CCHUNK, _LANES):
                idx = ib[r, pl.ds(c0, _LANES)]
                ob[r, pl.ds(c0, _LANES)] = plsc.load_gather(table_v, [idx])

    def in_slice(g):
        gc = g // _NRC
        gr = g % _NRC
        return values_hbm.at[pl.ds(gr * _RCHUNK, _RCHUNK),
                             pl.ds(c0w + gc * _CCHUNK, _CCHUNK)]

    def out_slice(g):
        gc = g // _NRC
        gr = g % _NRC
        return out_hbm.at[pl.ds(gr * _RCHUNK, _RCHUNK),
                          pl.ds(c0w + gc * _CCHUNK, _CCHUNK)]

    for b in range(_NB):
        pltpu.async_copy(in_slice(b), in_v.at[b], insem.at[b])

    for b in range(_NB):
        pltpu.make_async_copy(in_slice(b), in_v.at[b], insem.at[b]).wait()
        gather_chunk(b)
        pltpu.async_copy(out_v.at[b], out_slice(b), outsem.at[b])
        if b + _NB < _NCHUNK:
            pltpu.async_copy(in_slice(b + _NB), in_v.at[b], insem.at[b])

    @pl.loop(_NB, _NCHUNK, step=_NB)
    def _(g0):
        for b in range(_NB):
            g = g0 + b
            pltpu.make_async_copy(in_slice(g), in_v.at[b], insem.at[b]).wait()
            pltpu.make_async_copy(out_v.at[b], out_slice(g),
                                  outsem.at[b]).wait()
            gather_chunk(b)
            pltpu.async_copy(out_v.at[b], out_slice(g), outsem.at[b])

            @pl.when(g + _NB < _NCHUNK)
            def _():
                pltpu.async_copy(in_slice(g + _NB), in_v.at[b], insem.at[b])

    for b in range(_NB):
        pltpu.make_async_copy(out_v.at[b], out_slice(_NCHUNK - _NB + b),
                              outsem.at[b]).wait()


_lookup = _make()(_lookup_body)


def kernel(values, table):
    out_t = _lookup(jnp.swapaxes(values, 0, 1), table)
    return jnp.swapaxes(out_t, 0, 1)
